# Initial kernel scaffold; baseline (speedup 1.0000x reference)
import jax
import jax.numpy as jnp
from jax.experimental import pallas as pl


def kernel(x, pos, extra_x, edge_attr, ss, time, edge_index, batch, params):
    raise NotImplementedError("write your pallas kernel here")


# speedup vs baseline: 2.2627x; 2.2627x over previous
import functools

import jax
import jax.numpy as jnp
from jax import lax
from jax.experimental import pallas as pl
from jax.experimental.pallas import tpu as pltpu
from jax.experimental.pallas import tpu_sc as plsc

F32 = jnp.float32

NC = 2
NS = 16
NW = NC * NS

CHUNK = 128


def _silu(x):
    return x * jax.nn.sigmoid(x)


def _sc_gather(table, idx):
    rows, width = idx.shape[0], table.shape[1]
    assert rows % NW == 0
    rpw = rows // NW
    nfull, tail = rpw // CHUNK, rpw % CHUNK
    assert tail % 8 == 0
    mesh = plsc.VectorSubcoreMesh(core_axis_name="c", subcore_axis_name="s")

    @functools.partial(
        pl.kernel,
        out_type=jax.ShapeDtypeStruct((rows, width), F32),
        mesh=mesh,
        scratch_types=[
            pltpu.VMEM((CHUNK,), jnp.int32),
            pltpu.VMEM((CHUNK, width), F32),
            pltpu.VMEM((max(tail, 8),), jnp.int32),
            pltpu.VMEM((max(tail, 8), width), F32),
            pltpu.SemaphoreType.DMA,
        ],
    )
    def k(table_hbm, idx_hbm, out_hbm, idx_v, rows_v, idx_t, rows_t, sem):
        wid = lax.axis_index("s") * NC + lax.axis_index("c")
        base = wid * rpw

        def body(j, carry):
            off = base + j * CHUNK
            pltpu.sync_copy(idx_hbm.at[pl.ds(off, CHUNK)], idx_v)
            pltpu.async_copy(table_hbm.at[idx_v], rows_v, sem).wait()
            pltpu.sync_copy(rows_v, out_hbm.at[pl.ds(off, CHUNK)])
            return carry

        lax.fori_loop(0, nfull, body, 0)
        if tail:
            off = base + nfull * CHUNK
            pltpu.sync_copy(idx_hbm.at[pl.ds(off, tail)], idx_t)
            pltpu.async_copy(table_hbm.at[idx_t], rows_t, sem).wait()
            pltpu.sync_copy(rows_t, out_hbm.at[pl.ds(off, tail)])

    return k(table, idx)


def _sc_reldist(pos_t, src_pad, dst_pad, E):
    N = pos_t.shape[1]
    rpw = E // NW
    ngrp = (rpw + 15) // 16
    cap = ngrp * 16
    assert rpw % 8 == 0 and src_pad.shape[0] >= NW * rpw + (cap - rpw)
    mesh = plsc.VectorSubcoreMesh(core_axis_name="c", subcore_axis_name="s")

    @functools.partial(
        pl.kernel,
        out_type=jax.ShapeDtypeStruct((NW * rpw,), F32),
        mesh=mesh,
        scratch_types=[
            pltpu.VMEM((N,), F32),
            pltpu.VMEM((N,), F32),
            pltpu.VMEM((N,), F32),
            pltpu.VMEM((cap,), jnp.int32),
            pltpu.VMEM((cap,), jnp.int32),
            pltpu.VMEM((cap,), F32),
        ],
        compiler_params=pltpu.CompilerParams(needs_layout_passes=False),
    )
    def k(px_hbm, py_hbm, pz_hbm, src_hbm, dst_hbm, out_hbm, px_v, py_v,
          pz_v, si_v, di_v, out_v):
        wid = lax.axis_index("s") * NC + lax.axis_index("c")
        base = wid * rpw
        pltpu.sync_copy(px_hbm, px_v)
        pltpu.sync_copy(py_hbm, py_v)
        pltpu.sync_copy(pz_hbm, pz_v)
        pltpu.sync_copy(src_hbm.at[pl.ds(base, cap)], si_v)
        pltpu.sync_copy(dst_hbm.at[pl.ds(base, cap)], di_v)

        def body(j, carry):
            s = si_v[pl.ds(j * 16, 16)]
            d = di_v[pl.ds(j * 16, 16)]
            acc = jnp.zeros((16,), F32)
            for comp in (px_v, py_v, pz_v):
                r = plsc.load_gather(comp, [s]) - plsc.load_gather(comp, [d])
                acc = acc + r * r
            out_v[pl.ds(j * 16, 16)] = acc
            return carry

        lax.fori_loop(0, ngrp, body, 0)
        pltpu.sync_copy(out_v.at[pl.ds(0, rpw)], out_hbm.at[pl.ds(base, rpw)])

    return k(pos_t[0], pos_t[1], pos_t[2], src_pad, dst_pad)


def _sc_segsum(vals, idx, nseg_pad):
    rows, width = vals.shape
    assert rows % NW == 0
    rpw = rows // NW
    nfull, tail = rpw // CHUNK, rpw % CHUNK
    assert tail % 8 == 0 and nseg_pad % (8 * NS) == 0
    seg_pw = nseg_pad // NS
    zeros = jnp.zeros((nseg_pad, width), F32)
    mesh = plsc.VectorSubcoreMesh(core_axis_name="c", subcore_axis_name="s")

    @functools.partial(
        pl.kernel,
        out_type=jax.ShapeDtypeStruct((NC, nseg_pad, width), F32),
        mesh=mesh,
        scratch_types=[
            pltpu.VMEM((CHUNK,), jnp.int32),
            pltpu.VMEM((CHUNK, width), F32),
            pltpu.VMEM((max(tail, 8),), jnp.int32),
            pltpu.VMEM((max(tail, 8), width), F32),
            pltpu.VMEM_SHARED((nseg_pad, width), F32),
            pltpu.SemaphoreType.DMA,
        ],
    )
    def k(vals_hbm, idx_hbm, zeros_hbm, out_hbm, idx_v, val_v, idx_t, val_t,
          acc_sh, sem):
        cid = lax.axis_index("c")
        sid = lax.axis_index("s")
        wid = sid * NC + cid
        base = wid * rpw
        pltpu.sync_copy(zeros_hbm.at[pl.ds(sid * seg_pw, seg_pw)],
                        acc_sh.at[pl.ds(sid * seg_pw, seg_pw)])
        plsc.subcore_barrier()

        def body(j, carry):
            off = base + j * CHUNK
            pltpu.sync_copy(idx_hbm.at[pl.ds(off, CHUNK)], idx_v)
            pltpu.sync_copy(vals_hbm.at[pl.ds(off, CHUNK)], val_v)
            pltpu.sync_copy(val_v, acc_sh.at[idx_v], add=True)
            return carry

        lax.fori_loop(0, nfull, body, 0)
        if tail:
            off = base + nfull * CHUNK
            pltpu.sync_copy(idx_hbm.at[pl.ds(off, tail)], idx_t)
            pltpu.sync_copy(vals_hbm.at[pl.ds(off, tail)], val_t)
            pltpu.sync_copy(val_t, acc_sh.at[idx_t], add=True)
        plsc.subcore_barrier()
        pltpu.sync_copy(acc_sh.at[pl.ds(sid * seg_pw, seg_pw)],
                        out_hbm.at[cid, pl.ds(sid * seg_pw, seg_pw)])

    return k(vals, idx, zeros)


def _prep(time, ss, p):
    nlayers = len(p["layers"])

    def body(time_ref, ss_ref, wt1, bt1, wt2, bt2, ws1, bs1, ws2, bs2, wtl,
             btl, ss_e_ref, sc_ref, sh_ref):
        t = _silu(time_ref[...] @ wt1[...] + bt1[...]) @ wt2[...] + bt2[...]
        st = _silu(t)
        for l in range(nlayers):
            te = st @ wtl[l] + btl[l]
            sc_ref[l] = te[:, : te.shape[1] // 2]
            sh_ref[l] = te[:, te.shape[1] // 2:]
        ss_e_ref[...] = _silu(ss_ref[...] @ ws1[...] + bs1[...]) @ ws2[...] \
            + bs2[...]

    B = time.shape[0]
    N = ss.shape[0]
    D = p["time_mlp"][1]["W"].shape[1]
    wtl = jnp.stack([lp["time_lin"]["W"] for lp in p["layers"]])
    btl = jnp.stack([lp["time_lin"]["b"][None, :] for lp in p["layers"]])
    return pl.pallas_call(
        body,
        out_shape=[
            jax.ShapeDtypeStruct((N, D), F32),
            jax.ShapeDtypeStruct((nlayers, B, D), F32),
            jax.ShapeDtypeStruct((nlayers, B, D), F32),
        ],
    )(time, ss,
      p["time_mlp"][0]["W"], p["time_mlp"][0]["b"][None, :],
      p["time_mlp"][1]["W"], p["time_mlp"][1]["b"][None, :],
      p["ss_mlp"][0]["W"], p["ss_mlp"][0]["b"][None, :],
      p["ss_mlp"][1]["W"], p["ss_mlp"][1]["b"][None, :],
      wtl, btl)


BE = 640


def _edge_mlp(dist, gat, ea, lp, E, want_ea):
    D = (lp["edge_mlp1"]["W"].shape[0] - 1 - ea.shape[1]) // 2
    DE = ea.shape[1]
    H2 = lp["edge_mlp1"]["W"].shape[1]
    H = lp["edge_mlp2"]["W"].shape[1]
    nblk = E // BE
    w1 = lp["edge_mlp1"]["W"]

    def body(dist_r, fd, fs, ea_r, w1a, w1b, w1c, w1d, b1, w2, b2, *rest):
        h = _silu(fd[...] @ w1a[...] + fs[...] @ w1b[...]
                  + ea_r[...] @ w1c[...] + dist_r[...] * w1d[...] + b1[...])
        m = _silu(h @ w2[...] + b2[...])
        if want_ea:
            wu, bu, mij_ref, ea2_ref = rest
            ea2_ref[...] = _silu(m @ wu[...] + bu[...])
        else:
            mij_ref, = rest
        mij_ref[...] = m

    full = lambda r, c: pl.BlockSpec((r, c), lambda i: (0, 0))
    in_specs = [
        pl.BlockSpec((BE, 1), lambda i: (i, 0)),
        pl.BlockSpec((BE, D), lambda i: (i, 0)),
        pl.BlockSpec((BE, D), lambda i: (i + nblk, 0)),
        pl.BlockSpec((BE, DE), lambda i: (i, 0)),
        full(D, H2), full(D, H2), full(DE, H2), full(1, H2), full(1, H2),
        full(H2, H), full(1, H),
    ]
    args = [dist, gat, gat, ea,
            w1[:D], w1[D:2 * D], w1[2 * D:2 * D + DE], w1[2 * D + DE:],
            lp["edge_mlp1"]["b"][None, :],
            lp["edge_mlp2"]["W"], lp["edge_mlp2"]["b"][None, :]]
    out_specs = [pl.BlockSpec((BE, H), lambda i: (i, 0))]
    out_shape = [jax.ShapeDtypeStruct((E, H), F32)]
    if want_ea:
        in_specs += [full(H, DE), full(1, DE)]
        args += [lp["edge_upd"]["W"], lp["edge_upd"]["b"][None, :]]
        out_specs.append(pl.BlockSpec((BE, DE), lambda i: (i, 0)))
        out_shape.append(jax.ShapeDtypeStruct((E, DE), F32))
    out = pl.pallas_call(
        body, grid=(nblk,), in_specs=in_specs, out_specs=out_specs,
        out_shape=out_shape)(*args)
    return out if want_ea else out[0]


def _node_update(feats, part, scale, shift, batch_col, lp, final_args=None):
    N, D = feats.shape
    NB = scale.shape[0]

    def body(feats_ref, part_ref, sc_ref, sh_ref, batch_ref, w3a, w3b, b3,
             w4, b4, wf1, bf1, g, b, wf2, bf2, *rest):
        f0 = feats_ref[...]
        m_i = part_ref[0, :N, :] + part_ref[1, :N, :]
        u = _silu(f0 @ w3a[...] + m_i @ w3b[...] + b3[...])
        f1 = f0 + u @ w4[...] + b4[...]
        oh = (batch_ref[...] == lax.broadcasted_iota(jnp.int32, (N, NB), 1)
              ).astype(F32)
        f1 = f1 * (oh @ sc_ref[...] + 1.0) + oh @ sh_ref[...]
        f = _silu(f1 @ wf1[...] + bf1[...])
        gm = jnp.mean(f)
        gs = jnp.sqrt(jnp.mean((f - gm) ** 2))
        fn = (f - gm) / (gs + 1e-5) * g[...] + b[...]
        f2 = fn @ wf2[...] + bf2[...]
        if final_args is None:
            out_ref, = rest
            out_ref[...] = f2
        else:
            ss_e, wo, bo, out_ref = rest
            out_ref[...] = (f2 + ss_e[...]) @ wo[...] + bo[...]

    w3 = lp["node_mlp1"]["W"]
    args = [feats, part, scale, shift, batch_col,
            w3[:D], w3[D:], lp["node_mlp1"]["b"][None, :],
            lp["node_mlp2"]["W"], lp["node_mlp2"]["b"][None, :],
            lp["ff1"]["W"], lp["ff1"]["b"][None, :],
            lp["ln_g"][None, :], lp["ln_b"][None, :],
            lp["ff2"]["W"], lp["ff2"]["b"][None, :]]
    if final_args is None:
        out_shape = jax.ShapeDtypeStruct((N, D), F32)
    else:
        ss_e, po = final_args
        args += [ss_e, po["W"], po["b"][None, :]]
        out_shape = jax.ShapeDtypeStruct((N, po["W"].shape[1]), F32)
    return pl.pallas_call(body, out_shape=out_shape)(*args)


def kernel(x, pos, extra_x, edge_attr, ss, time, edge_index, batch, params):
    N = x.shape[0]
    E = edge_index.shape[1]
    src = edge_index[0].astype(jnp.int32)
    dst = edge_index[1].astype(jnp.int32)
    idx2 = jnp.concatenate([dst, src])
    batch_col = batch.astype(jnp.int32)[:, None]

    feats = jnp.concatenate([x, extra_x], axis=1)

    ss_e, sc, sh = _prep(time, ss, params)

    nseg_pad = ((N + 8 * NS - 1) // (8 * NS)) * (8 * NS)

    pad = -E % NW + 16
    dist = _sc_reldist(pos.T, jnp.pad(src, (0, pad)),
                       jnp.pad(dst, (0, pad)), E)[:, None]
    ea = edge_attr
    for l, lp in enumerate(params["layers"]):
        gat = _sc_gather(feats, idx2)
        if l == 0:
            m_ij, ea = _edge_mlp(dist, gat, ea, lp, E, want_ea=True)
        else:
            m_ij = _edge_mlp(dist, gat, ea, lp, E, want_ea=False)
        part = _sc_segsum(m_ij, dst, nseg_pad)
        final = None if l < len(params["layers"]) - 1 \
            else (ss_e, params["lin_out"])
        feats = _node_update(feats, part, sc[l], sh[l], batch_col, lp,
                             final_args=final)
    return feats


# speedup vs baseline: 2.7115x; 1.1984x over previous
import functools

import jax
import jax.numpy as jnp
from jax import lax
from jax.experimental import pallas as pl
from jax.experimental.pallas import tpu as pltpu
from jax.experimental.pallas import tpu_sc as plsc

F32 = jnp.float32

NC = 2
NS = 16
NW = NC * NS

CHUNK = 128


def _silu(x):
    return x * jax.nn.sigmoid(x)


NBUF = 6


def _sc_gather(table, idx):
    rows, width = idx.shape[0], table.shape[1]
    assert rows % NW == 0
    rpw = rows // NW
    nfull, tail = rpw // CHUNK, rpw % CHUNK
    ngrp = nfull // NBUF
    assert tail % 8 == 0 and nfull % NBUF == 0
    mesh = plsc.VectorSubcoreMesh(core_axis_name="c", subcore_axis_name="s")
    scratch = [pltpu.VMEM((rpw,), jnp.int32)]
    scratch += [pltpu.VMEM((CHUNK, width), F32) for _ in range(NBUF)]
    scratch += [pltpu.VMEM((max(tail, 8), width), F32)]
    scratch += [pltpu.SemaphoreType.DMA] * (2 * NBUF + 1)

    @functools.partial(
        pl.kernel,
        out_type=jax.ShapeDtypeStruct((rows, width), F32),
        mesh=mesh,
        scratch_types=scratch,
    )
    def k(table_hbm, idx_hbm, out_hbm, idx_v, *rest):
        bufs = rest[:NBUF]
        tbuf = rest[NBUF]
        gsem = rest[NBUF + 1:2 * NBUF + 1]
        wsem = rest[2 * NBUF + 1:3 * NBUF + 1]
        tsem = rest[-1]
        wid = lax.axis_index("s") * NC + lax.axis_index("c")
        base = wid * rpw
        pltpu.sync_copy(idx_hbm.at[pl.ds(base, rpw)], idx_v)
        for b in range(NBUF):
            pltpu.async_copy(table_hbm.at[idx_v.at[pl.ds(b * CHUNK, CHUNK)]],
                             bufs[b], gsem[b])

        def body(g, carry):
            for b in range(NBUF):
                c = g * NBUF + b
                pltpu.make_async_copy(
                    table_hbm.at[idx_v.at[pl.ds(0, CHUNK)]], bufs[b],
                    gsem[b]).wait()
                pltpu.async_copy(
                    bufs[b], out_hbm.at[pl.ds(base + c * CHUNK, CHUNK)],
                    wsem[b])

            @pl.when(g < ngrp - 1)
            def _():
                for b in range(NBUF):
                    c = (g + 1) * NBUF + b
                    pltpu.make_async_copy(
                        bufs[b], out_hbm.at[pl.ds(base, CHUNK)],
                        wsem[b]).wait()
                    pltpu.async_copy(
                        table_hbm.at[idx_v.at[pl.ds(c * CHUNK, CHUNK)]],
                        bufs[b], gsem[b])

            return carry

        lax.fori_loop(0, ngrp, body, 0)
        for b in range(NBUF):
            pltpu.make_async_copy(bufs[b], out_hbm.at[pl.ds(base, CHUNK)],
                                  wsem[b]).wait()
        if tail:
            off = base + nfull * CHUNK
            pltpu.async_copy(
                table_hbm.at[idx_v.at[pl.ds(nfull * CHUNK, tail)]],
                tbuf.at[pl.ds(0, tail)], tsem).wait()
            pltpu.sync_copy(tbuf.at[pl.ds(0, tail)],
                            out_hbm.at[pl.ds(off, tail)])

    return k(table, idx)


def _sc_reldist(pos_t, src_pad, dst_pad, E):
    N = pos_t.shape[1]
    rpw = E // NW
    ngrp = (rpw + 15) // 16
    cap = ngrp * 16
    assert rpw % 8 == 0 and src_pad.shape[0] >= NW * rpw + (cap - rpw)
    mesh = plsc.VectorSubcoreMesh(core_axis_name="c", subcore_axis_name="s")

    @functools.partial(
        pl.kernel,
        out_type=jax.ShapeDtypeStruct((NW * rpw,), F32),
        mesh=mesh,
        scratch_types=[
            pltpu.VMEM((N,), F32),
            pltpu.VMEM((N,), F32),
            pltpu.VMEM((N,), F32),
            pltpu.VMEM((cap,), jnp.int32),
            pltpu.VMEM((cap,), jnp.int32),
            pltpu.VMEM((cap,), F32),
        ],
        compiler_params=pltpu.CompilerParams(needs_layout_passes=False),
    )
    def k(px_hbm, py_hbm, pz_hbm, src_hbm, dst_hbm, out_hbm, px_v, py_v,
          pz_v, si_v, di_v, out_v):
        wid = lax.axis_index("s") * NC + lax.axis_index("c")
        base = wid * rpw
        pltpu.sync_copy(px_hbm, px_v)
        pltpu.sync_copy(py_hbm, py_v)
        pltpu.sync_copy(pz_hbm, pz_v)
        pltpu.sync_copy(src_hbm.at[pl.ds(base, cap)], si_v)
        pltpu.sync_copy(dst_hbm.at[pl.ds(base, cap)], di_v)

        def body(j, carry):
            s = si_v[pl.ds(j * 16, 16)]
            d = di_v[pl.ds(j * 16, 16)]
            acc = jnp.zeros((16,), F32)
            for comp in (px_v, py_v, pz_v):
                r = plsc.load_gather(comp, [s]) - plsc.load_gather(comp, [d])
                acc = acc + r * r
            out_v[pl.ds(j * 16, 16)] = acc
            return carry

        lax.fori_loop(0, ngrp, body, 0)
        pltpu.sync_copy(out_v.at[pl.ds(0, rpw)], out_hbm.at[pl.ds(base, rpw)])

    return k(pos_t[0], pos_t[1], pos_t[2], src_pad, dst_pad)


def _sc_segsum(vals, idx, nseg_pad):
    rows, width = vals.shape
    assert rows % NW == 0
    rpw = rows // NW
    nfull, tail = rpw // CHUNK, rpw % CHUNK
    assert tail % 8 == 0 and nseg_pad % (8 * NS) == 0
    seg_pw = nseg_pad // NS
    zeros = jnp.zeros((nseg_pad, width), F32)
    mesh = plsc.VectorSubcoreMesh(core_axis_name="c", subcore_axis_name="s")

    assert nfull % 2 == 1
    npair = (nfull - 1) // 2

    @functools.partial(
        pl.kernel,
        out_type=jax.ShapeDtypeStruct((NC, nseg_pad, width), F32),
        mesh=mesh,
        scratch_types=[
            pltpu.VMEM((CHUNK,), jnp.int32),
            pltpu.VMEM((CHUNK, width), F32),
            pltpu.VMEM((CHUNK,), jnp.int32),
            pltpu.VMEM((CHUNK, width), F32),
            pltpu.VMEM((max(tail, 8),), jnp.int32),
            pltpu.VMEM((max(tail, 8), width), F32),
            pltpu.VMEM_SHARED((nseg_pad, width), F32),
            pltpu.SemaphoreType.DMA,
            pltpu.SemaphoreType.DMA,
        ],
    )
    def k(vals_hbm, idx_hbm, zeros_hbm, out_hbm, idx_a, val_a, idx_b, val_b,
          idx_t, val_t, acc_sh, sem_a, sem_b):
        cid = lax.axis_index("c")
        sid = lax.axis_index("s")
        wid = sid * NC + cid
        base = wid * rpw

        def load(c, ib, vb, sem):
            off = base + c * CHUNK
            pltpu.async_copy(idx_hbm.at[pl.ds(off, CHUNK)], ib, sem)
            pltpu.async_copy(vals_hbm.at[pl.ds(off, CHUNK)], vb, sem)

        def drain(ib, vb, sem):
            pltpu.make_async_copy(idx_hbm.at[pl.ds(base, CHUNK)], ib,
                                  sem).wait()
            pltpu.make_async_copy(vals_hbm.at[pl.ds(base, CHUNK)], vb,
                                  sem).wait()

        pltpu.sync_copy(zeros_hbm.at[pl.ds(sid * seg_pw, seg_pw)],
                        acc_sh.at[pl.ds(sid * seg_pw, seg_pw)])
        load(0, idx_a, val_a, sem_a)
        plsc.subcore_barrier()

        def body(p, carry):
            load(2 * p + 1, idx_b, val_b, sem_b)
            drain(idx_a, val_a, sem_a)
            pltpu.sync_copy(val_a, acc_sh.at[idx_a], add=True)
            load(2 * p + 2, idx_a, val_a, sem_a)
            drain(idx_b, val_b, sem_b)
            pltpu.sync_copy(val_b, acc_sh.at[idx_b], add=True)
            return carry

        lax.fori_loop(0, npair, body, 0)
        drain(idx_a, val_a, sem_a)
        pltpu.sync_copy(val_a, acc_sh.at[idx_a], add=True)
        if tail:
            off = base + nfull * CHUNK
            pltpu.sync_copy(idx_hbm.at[pl.ds(off, tail)], idx_t)
            pltpu.sync_copy(vals_hbm.at[pl.ds(off, tail)], val_t)
            pltpu.sync_copy(val_t, acc_sh.at[idx_t], add=True)
        plsc.subcore_barrier()
        pltpu.sync_copy(acc_sh.at[pl.ds(sid * seg_pw, seg_pw)],
                        out_hbm.at[cid, pl.ds(sid * seg_pw, seg_pw)])

    return k(vals, idx, zeros)


def _prep(time, ss, p):
    nlayers = len(p["layers"])

    def body(time_ref, ss_ref, wt1, bt1, wt2, bt2, ws1, bs1, ws2, bs2, wtl,
             btl, ss_e_ref, sc_ref, sh_ref):
        t = _silu(time_ref[...] @ wt1[...] + bt1[...]) @ wt2[...] + bt2[...]
        st = _silu(t)
        for l in range(nlayers):
            te = st @ wtl[l] + btl[l]
            sc_ref[l] = te[:, : te.shape[1] // 2]
            sh_ref[l] = te[:, te.shape[1] // 2:]
        ss_e_ref[...] = _silu(ss_ref[...] @ ws1[...] + bs1[...]) @ ws2[...] \
            + bs2[...]

    B = time.shape[0]
    N = ss.shape[0]
    D = p["time_mlp"][1]["W"].shape[1]
    wtl = jnp.stack([lp["time_lin"]["W"] for lp in p["layers"]])
    btl = jnp.stack([lp["time_lin"]["b"][None, :] for lp in p["layers"]])
    return pl.pallas_call(
        body,
        out_shape=[
            jax.ShapeDtypeStruct((N, D), F32),
            jax.ShapeDtypeStruct((nlayers, B, D), F32),
            jax.ShapeDtypeStruct((nlayers, B, D), F32),
        ],
    )(time, ss,
      p["time_mlp"][0]["W"], p["time_mlp"][0]["b"][None, :],
      p["time_mlp"][1]["W"], p["time_mlp"][1]["b"][None, :],
      p["ss_mlp"][0]["W"], p["ss_mlp"][0]["b"][None, :],
      p["ss_mlp"][1]["W"], p["ss_mlp"][1]["b"][None, :],
      wtl, btl)


BE = 640


def _edge_mlp(dist, gat, ea, lp, E, want_ea):
    D = (lp["edge_mlp1"]["W"].shape[0] - 1 - ea.shape[1]) // 2
    DE = ea.shape[1]
    H2 = lp["edge_mlp1"]["W"].shape[1]
    H = lp["edge_mlp2"]["W"].shape[1]
    nblk = E // BE
    w1 = lp["edge_mlp1"]["W"]

    def body(dist_r, fd, fs, ea_r, w1a, w1b, w1c, w1d, b1, w2, b2, *rest):
        h = _silu(fd[...] @ w1a[...] + fs[...] @ w1b[...]
                  + ea_r[...] @ w1c[...] + dist_r[...] * w1d[...] + b1[...])
        m = _silu(h @ w2[...] + b2[...])
        if want_ea:
            wu, bu, mij_ref, ea2_ref = rest
            ea2_ref[...] = _silu(m @ wu[...] + bu[...])
        else:
            mij_ref, = rest
        mij_ref[...] = m

    full = lambda r, c: pl.BlockSpec((r, c), lambda i: (0, 0))
    in_specs = [
        pl.BlockSpec((BE, 1), lambda i: (i, 0)),
        pl.BlockSpec((BE, D), lambda i: (i, 0)),
        pl.BlockSpec((BE, D), lambda i: (i + nblk, 0)),
        pl.BlockSpec((BE, DE), lambda i: (i, 0)),
        full(D, H2), full(D, H2), full(DE, H2), full(1, H2), full(1, H2),
        full(H2, H), full(1, H),
    ]
    args = [dist, gat, gat, ea,
            w1[:D], w1[D:2 * D], w1[2 * D:2 * D + DE], w1[2 * D + DE:],
            lp["edge_mlp1"]["b"][None, :],
            lp["edge_mlp2"]["W"], lp["edge_mlp2"]["b"][None, :]]
    out_specs = [pl.BlockSpec((BE, H), lambda i: (i, 0))]
    out_shape = [jax.ShapeDtypeStruct((E, H), F32)]
    if want_ea:
        in_specs += [full(H, DE), full(1, DE)]
        args += [lp["edge_upd"]["W"], lp["edge_upd"]["b"][None, :]]
        out_specs.append(pl.BlockSpec((BE, DE), lambda i: (i, 0)))
        out_shape.append(jax.ShapeDtypeStruct((E, DE), F32))
    out = pl.pallas_call(
        body, grid=(nblk,), in_specs=in_specs, out_specs=out_specs,
        out_shape=out_shape)(*args)
    return out if want_ea else out[0]


def _node_update(feats, part, scale, shift, batch_col, lp, final_args=None):
    N, D = feats.shape
    NB = scale.shape[0]

    def body(feats_ref, part_ref, sc_ref, sh_ref, batch_ref, w3a, w3b, b3,
             w4, b4, wf1, bf1, g, b, wf2, bf2, *rest):
        f0 = feats_ref[...]
        m_i = part_ref[0, :N, :] + part_ref[1, :N, :]
        u = _silu(f0 @ w3a[...] + m_i @ w3b[...] + b3[...])
        f1 = f0 + u @ w4[...] + b4[...]
        oh = (batch_ref[...] == lax.broadcasted_iota(jnp.int32, (N, NB), 1)
              ).astype(F32)
        f1 = f1 * (oh @ sc_ref[...] + 1.0) + oh @ sh_ref[...]
        f = _silu(f1 @ wf1[...] + bf1[...])
        gm = jnp.mean(f)
        gs = jnp.sqrt(jnp.mean((f - gm) ** 2))
        fn = (f - gm) / (gs + 1e-5) * g[...] + b[...]
        f2 = fn @ wf2[...] + bf2[...]
        if final_args is None:
            out_ref, = rest
            out_ref[...] = f2
        else:
            ss_e, wo, bo, out_ref = rest
            out_ref[...] = (f2 + ss_e[...]) @ wo[...] + bo[...]

    w3 = lp["node_mlp1"]["W"]
    args = [feats, part, scale, shift, batch_col,
            w3[:D], w3[D:], lp["node_mlp1"]["b"][None, :],
            lp["node_mlp2"]["W"], lp["node_mlp2"]["b"][None, :],
            lp["ff1"]["W"], lp["ff1"]["b"][None, :],
            lp["ln_g"][None, :], lp["ln_b"][None, :],
            lp["ff2"]["W"], lp["ff2"]["b"][None, :]]
    if final_args is None:
        out_shape = jax.ShapeDtypeStruct((N, D), F32)
    else:
        ss_e, po = final_args
        args += [ss_e, po["W"], po["b"][None, :]]
        out_shape = jax.ShapeDtypeStruct((N, po["W"].shape[1]), F32)
    return pl.pallas_call(body, out_shape=out_shape)(*args)


def kernel(x, pos, extra_x, edge_attr, ss, time, edge_index, batch, params):
    N = x.shape[0]
    E = edge_index.shape[1]
    src = edge_index[0].astype(jnp.int32)
    dst = edge_index[1].astype(jnp.int32)
    idx2 = jnp.concatenate([dst, src])
    batch_col = batch.astype(jnp.int32)[:, None]

    feats = jnp.concatenate([x, extra_x], axis=1)

    ss_e, sc, sh = _prep(time, ss, params)

    nseg_pad = ((N + 8 * NS - 1) // (8 * NS)) * (8 * NS)

    pad = -E % NW + 16
    dist = _sc_reldist(pos.T, jnp.pad(src, (0, pad)),
                       jnp.pad(dst, (0, pad)), E)[:, None]
    ea = edge_attr
    for l, lp in enumerate(params["layers"]):
        gat = _sc_gather(feats, idx2)
        if l == 0:
            m_ij, ea = _edge_mlp(dist, gat, ea, lp, E, want_ea=True)
        else:
            m_ij = _edge_mlp(dist, gat, ea, lp, E, want_ea=False)
        part = _sc_segsum(m_ij, dst, nseg_pad)
        final = None if l < len(params["layers"]) - 1 \
            else (ss_e, params["lin_out"])
        feats = _node_update(feats, part, sc[l], sh[l], batch_col, lp,
                             final_args=final)
    return feats
